# baseline (speedup 1.0000x reference)
import functools

import jax
import jax.numpy as jnp
from jax import lax
from jax.experimental import pallas as pl
from jax.experimental.pallas import tpu as pltpu
from jax.experimental.pallas import tpu_sc as plsc

BATCH = 16384
LANES = 128

_info = plsc.get_sparse_core_info()
NC, NS = _info.num_cores, _info.num_subcores
NW = NC * NS
BPW = BATCH // NW
CHUNK = 128
NCHUNK = BPW // CHUNK

_sc_mesh = plsc.VectorSubcoreMesh(core_axis_name="c", subcore_axis_name="s")


@functools.partial(
    pl.kernel,
    out_type=jax.ShapeDtypeStruct((BATCH, LANES), jnp.float32),
    mesh=_sc_mesh,
    scratch_types=[
        pltpu.VMEM((BPW,), jnp.int32),
        pltpu.VMEM((BPW,), jnp.int32),
        pltpu.VMEM((BPW,), jnp.int32),
        pltpu.VMEM((CHUNK, LANES), jnp.float32),
        pltpu.VMEM((CHUNK, LANES), jnp.float32),
        pltpu.VMEM((CHUNK, LANES), jnp.float32),
        pltpu.VMEM((CHUNK, LANES), jnp.float32),
        pltpu.SemaphoreType.DMA,
        pltpu.SemaphoreType.DMA,
    ],
)
def _sc_gather(sub_idx_hbm, trade_idx_hbm, cert_idx_hbm,
               sub_tab_hbm, combo_tab_hbm,
               x_out,
               sub_idx_v, trade_idx_v, combo_idx_v,
               sub_rows0, sub_rows1, combo_rows0, combo_rows1,
               sem_g, sem_w):
    wid = lax.axis_index("s") * NC + lax.axis_index("c")
    base = wid * BPW

    pltpu.sync_copy(sub_idx_hbm.at[pl.ds(base, BPW)], sub_idx_v)
    pltpu.sync_copy(trade_idx_hbm.at[pl.ds(base, BPW)], trade_idx_v)
    pltpu.sync_copy(cert_idx_hbm.at[pl.ds(base, BPW)], combo_idx_v)

    for k in range(BPW // 16):
        sl = pl.ds(k * 16, 16)
        combo_idx_v[sl] = trade_idx_v[sl] * 9 + combo_idx_v[sl]

    sub_bufs = [sub_rows0, sub_rows1]
    combo_bufs = [combo_rows0, combo_rows1]
    gathers = [None] * NCHUNK
    writes = [None] * NCHUNK

    def fire(j):
        b = j % 2
        isl = pl.ds(j * CHUNK, CHUNK)
        gathers[j] = (
            pltpu.async_copy(sub_tab_hbm.at[sub_idx_v.at[isl]],
                             sub_bufs[b], sem_g),
            pltpu.async_copy(combo_tab_hbm.at[combo_idx_v.at[isl]],
                             combo_bufs[b], sem_g),
        )

    fire(0)
    for j in range(NCHUNK):
        if j + 1 < NCHUNK:
            if j - 1 >= 0:
                writes[j - 1].wait()
            fire(j + 1)
        for g in gathers[j]:
            g.wait()
        b = j % 2
        sb, cb = sub_bufs[b], combo_bufs[b]

        def add_row(r, _):
            for k in (2, 3):
                sl = pl.ds(k * 16, 16)
                sb.at[r][sl] = sb.at[r][sl] + cb.at[r][sl]
            return 0

        lax.fori_loop(0, CHUNK, add_row, 0)
        writes[j] = pltpu.async_copy(
            sb, x_out.at[pl.ds(base + j * CHUNK, CHUNK)], sem_w)
    writes[NCHUNK - 2].wait()
    writes[NCHUNK - 1].wait()


def _mlp_body(x_ref, w1_ref, b1_ref, w2_ref, b2_ref, w3_ref, b3_ref, out_ref):
    x = x_ref[:, :56].astype(jnp.bfloat16)
    w1 = w1_ref[...].astype(jnp.bfloat16)
    w2 = w2_ref[...].astype(jnp.bfloat16)
    w3 = w3_ref[...].astype(jnp.bfloat16)
    h = jnp.dot(x, w1, preferred_element_type=jnp.float32) + b1_ref[...]
    h = jnp.maximum(h, 0.0).astype(jnp.bfloat16)
    h = jnp.dot(h, w2, preferred_element_type=jnp.float32) + b2_ref[...]
    h = jnp.maximum(h, 0.0).astype(jnp.bfloat16)
    out_ref[...] = jnp.dot(h, w3, preferred_element_type=jnp.float32) + b3_ref[...]


B_BLK = 2048


def _mlp(x, w1, b1, w2, b2, w3, b3):
    full = lambda shape: pl.BlockSpec(shape, lambda i: tuple(0 for _ in shape))
    return pl.pallas_call(
        _mlp_body,
        grid=(BATCH // B_BLK,),
        in_specs=[
            pl.BlockSpec((B_BLK, LANES), lambda i: (i, 0)),
            full((56, 512)),
            full((512,)),
            full((512, 128)),
            full((128,)),
            full((128, 64)),
            full((64,)),
        ],
        out_specs=pl.BlockSpec((B_BLK, 64), lambda i: (i, 0)),
        out_shape=jax.ShapeDtypeStruct((BATCH, 64), jnp.float32),
    )(x, w1, b1, w2, b2, w3, b3)


def kernel(subcontractor_id, primary_trade_id, certification_id,
           sub_table, trade_table, cert_table,
           W1, b1, W2, b2, W3, b3):
    sub_idx = subcontractor_id.astype(jnp.int32)
    trade_idx = primary_trade_id.astype(jnp.int32)
    cert_idx = certification_id.astype(jnp.int32)

    sub_tab_p = jnp.pad(sub_table, ((0, 0), (0, LANES - 32)))
    n_trade, n_cert = trade_table.shape[0], cert_table.shape[0]
    combo_tab = jnp.concatenate([
        jnp.zeros((n_trade, n_cert, 32), jnp.float32),
        jnp.broadcast_to(trade_table[:, None, :], (n_trade, n_cert, 16)),
        jnp.broadcast_to(cert_table[None, :, :], (n_trade, n_cert, 8)),
        jnp.zeros((n_trade, n_cert, LANES - 56), jnp.float32),
    ], axis=-1).reshape(n_trade * n_cert, LANES)

    x = _sc_gather(sub_idx, trade_idx, cert_idx, sub_tab_p, combo_tab)
    return _mlp(x, W1, b1, W2, b2, W3, b3)
